# baseline (speedup 1.0000x reference)
import functools

import jax
import jax.numpy as jnp
from jax import lax
from jax.experimental import pallas as pl
from jax.experimental.pallas import tpu as pltpu
from jax.experimental.pallas import tpu_sc as plsc

B = 4096
F = 26
V = 100000
K = 64
H1 = 512
H2 = 256

NC = 2
NS = 16
NW = NC * NS
CHB = B // NW


def _sc_gather(w0, idx_t):
    mesh = plsc.VectorSubcoreMesh(core_axis_name="c", subcore_axis_name="s")

    @functools.partial(
        pl.kernel,
        mesh=mesh,
        out_type=jax.ShapeDtypeStruct((B, F, K), jnp.float32),
        scratch_types=[
            pltpu.VMEM((CHB,), jnp.int32),
            pltpu.VMEM((CHB, K), jnp.float32),
            pltpu.SemaphoreType.DMA,
        ],
        compiler_params=pltpu.CompilerParams(use_tc_tiling_on_sc=False),
    )
    def gather_k(w0_hbm, idx_hbm, out_hbm, idx_v, rows_v, sem):
        wid = lax.axis_index("s") * NC + lax.axis_index("c")
        b0 = wid * CHB

        def body(f, carry):
            pltpu.sync_copy(idx_hbm.at[pl.ds(f * B + b0, CHB)], idx_v)
            pltpu.async_copy(w0_hbm.at[f].at[idx_v], rows_v, sem).wait()
            pltpu.sync_copy(rows_v, out_hbm.at[pl.ds(b0, CHB), f])
            return carry

        lax.fori_loop(0, F, body, 0)

    return gather_k(w0, idx_t)


def _tc_mlp(x, b0f, w1e, b1, w2, b2, w3, b3):
    bB = 512

    def mlp_k(x_ref, b0_ref, w1_ref, b1_ref, w2_ref, b2_ref, w3_ref, b3_ref,
              o_ref):
        l = jnp.tanh(x_ref[...] + b0_ref[...])
        h1 = jnp.maximum(
            jnp.dot(l, w1_ref[...], preferred_element_type=jnp.float32)
            + b1_ref[...], 0.0)
        h2 = jnp.maximum(
            jnp.dot(h1, w2_ref[...], preferred_element_type=jnp.float32)
            + b2_ref[...], 0.0)
        o = jnp.dot(h2, w3_ref[...], preferred_element_type=jnp.float32)
        o_ref[...] = jax.nn.sigmoid(o + b3_ref[...])

    return pl.pallas_call(
        mlp_k,
        grid=(B // bB,),
        in_specs=[
            pl.BlockSpec((bB, F * K), lambda i: (i, 0)),
            pl.BlockSpec((1, F * K), lambda i: (0, 0)),
            pl.BlockSpec((F * K, H1), lambda i: (0, 0)),
            pl.BlockSpec((1, H1), lambda i: (0, 0)),
            pl.BlockSpec((H1, H2), lambda i: (0, 0)),
            pl.BlockSpec((1, H2), lambda i: (0, 0)),
            pl.BlockSpec((H2, 1), lambda i: (0, 0)),
            pl.BlockSpec((1, 1), lambda i: (0, 0)),
        ],
        out_specs=pl.BlockSpec((bB, 1), lambda i: (i, 0)),
        out_shape=jax.ShapeDtypeStruct((B, 1), jnp.float32),
    )(x, b0f, w1e, b1, w2, b2, w3, b3)


def kernel(indices, W0, b0, w1, k1, b1, w2, b2, w3, b3):
    idx_t = indices.astype(jnp.int32).T.reshape(F * B)
    emb = _sc_gather(W0, idx_t)
    w1e = w1 + jnp.repeat(k1, K, axis=0)
    out = _tc_mlp(emb.reshape(B, F * K), b0.reshape(1, F * K), w1e,
                  b1.reshape(1, H1), w2, b2.reshape(1, H2), w3,
                  b3.reshape(1, 1))
    return out.reshape(-1)


# speedup vs baseline: 1.5929x; 1.5929x over previous
import functools

import jax
import jax.numpy as jnp
from jax import lax
from jax.experimental import pallas as pl
from jax.experimental.pallas import tpu as pltpu
from jax.experimental.pallas import tpu_sc as plsc

B = 4096
F = 26
V = 100000
K = 64
H1 = 512
H2 = 256

NC = 2
NS = 16
NW = NC * NS
ROWS_PER_W = B * F // NW
CH = 128
NCH = ROWS_PER_W // CH


def _sc_gather(w0, idx):
    mesh = plsc.VectorSubcoreMesh(core_axis_name="c", subcore_axis_name="s")

    @functools.partial(
        pl.kernel,
        mesh=mesh,
        out_type=jax.ShapeDtypeStruct((B * F, K), jnp.float32),
        scratch_types=[
            pltpu.VMEM((CH,), jnp.int32),
            pltpu.VMEM((CH, K), jnp.float32),
            pltpu.SemaphoreType.DMA,
        ],
    )
    def gather_k(w0_hbm, idx_hbm, out_hbm, idx_v, rows_v, sem):
        wid = lax.axis_index("s") * NC + lax.axis_index("c")
        base = wid * ROWS_PER_W

        def chunk(j, carry):
            r0 = base + j * CH
            pltpu.sync_copy(idx_hbm.at[pl.ds(r0, CH)], idx_v)

            def fire(g, c):
                vec = idx_v[pl.ds(g * 16, 16)]
                for l in range(16):
                    i = g * 16 + l
                    v = vec[l]
                    f = lax.rem(r0 + i, F)
                    pltpu.async_copy(
                        w0_hbm.at[f].at[pl.ds(v, 1)],
                        rows_v.at[pl.ds(i, 1)], sem)
                return c

            lax.fori_loop(0, CH // 16, fire, 0)

            def drain(i, c):
                pltpu.make_async_copy(
                    w0_hbm.at[0].at[pl.ds(0, 1)],
                    rows_v.at[pl.ds(0, 1)], sem).wait()
                return c

            lax.fori_loop(0, CH, drain, 0)
            pltpu.sync_copy(rows_v, out_hbm.at[pl.ds(r0, CH)])
            return carry

        lax.fori_loop(0, NCH, chunk, 0)

    return gather_k(w0, idx)


def _tc_mlp(x, b0f, w1e, b1, w2, b2, w3, b3):
    bB = 512

    def mlp_k(x_ref, b0_ref, w1_ref, b1_ref, w2_ref, b2_ref, w3_ref, b3_ref,
              o_ref):
        l = jnp.tanh(x_ref[...] + b0_ref[...])
        h1 = jnp.maximum(
            jnp.dot(l, w1_ref[...], preferred_element_type=jnp.float32)
            + b1_ref[...], 0.0)
        h2 = jnp.maximum(
            jnp.dot(h1, w2_ref[...], preferred_element_type=jnp.float32)
            + b2_ref[...], 0.0)
        o = jnp.dot(h2, w3_ref[...], preferred_element_type=jnp.float32)
        o_ref[...] = jax.nn.sigmoid(o + b3_ref[...])

    return pl.pallas_call(
        mlp_k,
        grid=(B // bB,),
        in_specs=[
            pl.BlockSpec((bB, F * K), lambda i: (i, 0)),
            pl.BlockSpec((1, F * K), lambda i: (0, 0)),
            pl.BlockSpec((F * K, H1), lambda i: (0, 0)),
            pl.BlockSpec((1, H1), lambda i: (0, 0)),
            pl.BlockSpec((H1, H2), lambda i: (0, 0)),
            pl.BlockSpec((1, H2), lambda i: (0, 0)),
            pl.BlockSpec((H2, 1), lambda i: (0, 0)),
            pl.BlockSpec((1, 1), lambda i: (0, 0)),
        ],
        out_specs=pl.BlockSpec((bB, 1), lambda i: (i, 0)),
        out_shape=jax.ShapeDtypeStruct((B, 1), jnp.float32),
    )(x, b0f, w1e, b1, w2, b2, w3, b3)


def kernel(indices, W0, b0, w1, k1, b1, w2, b2, w3, b3):
    idx = indices.astype(jnp.int32).reshape(B * F)
    emb = _sc_gather(W0, idx).reshape(B, F * K)
    w1e = w1 + jnp.repeat(k1, K, axis=0)
    out = _tc_mlp(emb, b0.reshape(1, F * K), w1e,
                  b1.reshape(1, H1), w2, b2.reshape(1, H2), w3,
                  b3.reshape(1, 1))
    return out.reshape(-1)
